# baseline (speedup 1.0000x reference)
import functools

import jax
import jax.numpy as jnp
from jax import lax
from jax.experimental import pallas as pl
from jax.experimental.pallas import tpu as pltpu
from jax.experimental.pallas import tpu_sc as plsc

N = 10000
E = 320000
EP = 327680
F = 128
HID = 32
MAXZ = 119

_NS = 16
_NC = 2
_NTILE = _NS * _NC
_EA = E // _NS
_EAP = 20096
_AROWS = _EAP // 128
_EB = E // _NTILE
_CH = 2000
_NCH = _EB // _CH
_VCH = _CH // 16
_CNTP = N + 16
_DUMMY = N

@functools.cache
def _sc_edge_streams_kernel():
    mesh = plsc.VectorSubcoreMesh(core_axis_name="c", subcore_axis_name="s",
                                  num_cores=_NC, num_subcores=_NS)
    return pl.kernel(
        _sc_body,
        out_type=[jax.ShapeDtypeStruct((EP,), jnp.float32)] * 4,
        mesh=mesh,
        scratch_types=[
            pltpu.VMEM_SHARED((_CNTP,), jnp.float32),
            pltpu.VMEM((_EAP,), jnp.int32),
            pltpu.VMEM((_AROWS, 128), jnp.int32),
            pltpu.VMEM((128,), jnp.float32),
            pltpu.VMEM((_CNTP,), jnp.float32),
            pltpu.VMEM((N,), jnp.int32),
            pltpu.VMEM((N,), jnp.int32),
            pltpu.VMEM((2 * _CH,), jnp.int32),
            pltpu.VMEM((2 * _CH,), jnp.int32),
            pltpu.VMEM((2 * 3 * _CH,), jnp.float32),
            pltpu.VMEM((2 * _CH,), jnp.float32),
            pltpu.VMEM((2 * _CH,), jnp.float32),
            pltpu.VMEM((2 * _CH,), jnp.float32),
            pltpu.VMEM((2 * _CH,), jnp.float32),
            pltpu.VMEM((EP - E,), jnp.float32),
            pltpu.SemaphoreType.DMA,
            pltpu.SemaphoreType.DMA,
            pltpu.SemaphoreType.DMA,
        ],
        compiler_params=pltpu.CompilerParams(needs_layout_passes=False),
    )


def _sc_body(rv_hbm, send_hbm, recv_hbm, num_hbm, gr_hbm,
                     rn2_hbm, z_hbm, gi_hbm, w_hbm,
                     cnt_sh, buf1d, idx2, ones_row, winv_v, numbers_v,
                     graph_v, send_v, recv_v, rv_v, rn2_v, z_v, gi_v, w_v,
                     zpad_v, semA, semL, semS):
    c = lax.axis_index("c")
    s = lax.axis_index("s")
    wid = s * _NC + c

    ones16 = jnp.ones((16,), jnp.float32)
    for k in range(8):
        ones_row[pl.ds(k * 16, 16)] = ones16
    dummy16 = jnp.full((16,), _DUMMY, jnp.int32)
    for k in range((_EAP - _EA) // 16):
        buf1d[pl.ds(_EA + k * 16, 16)] = dummy16

    @pl.when(wid == 0)
    def _zero_tails():
        z16f = jnp.zeros((16,), jnp.float32)

        def zb(i, _):
            zpad_v[pl.ds(i * 16, 16)] = z16f
            return 0

        lax.fori_loop(0, (EP - E) // 16, zb, 0)
        pltpu.sync_copy(zpad_v, rn2_hbm.at[pl.ds(E, EP - E)])
        pltpu.sync_copy(zpad_v, z_hbm.at[pl.ds(E, EP - E)])
        pltpu.sync_copy(zpad_v, gi_hbm.at[pl.ds(E, EP - E)])
        pltpu.sync_copy(zpad_v, w_hbm.at[pl.ds(E, EP - E)])

    @pl.when(s == 0)
    def _zero_cnt():
        z16 = jnp.zeros((16,), jnp.float32)

        def zbody(i, _):
            winv_v[pl.ds(i * 16, 16)] = z16
            return 0

        lax.fori_loop(0, _CNTP // 16, zbody, 0)
        pltpu.sync_copy(winv_v, cnt_sh)

    pltpu.sync_copy(recv_hbm.at[pl.ds(s * _EA, _EA)], buf1d.at[pl.ds(0, _EA)])

    @plsc.parallel_loop(0, _EAP // 16, unroll=8)
    def repack(k):
        idx2[k // 8, pl.ds((k % 8) * 16, 16)] = buf1d[pl.ds(k * 16, 16)]

    plsc.subcore_barrier()

    def fire(j, _):
        pltpu.make_async_copy(ones_row, cnt_sh.at[idx2.at[j]], semA).start(add=True)

        @pl.when(j >= 16)
        def _():
            pltpu.make_async_copy(ones_row, cnt_sh.at[idx2.at[0]], semA).wait()

        return 0

    lax.fori_loop(0, _AROWS, fire, 0)

    def drain(j, _):
        pltpu.make_async_copy(ones_row, cnt_sh.at[idx2.at[0]], semA).wait()
        return 0

    lax.fori_loop(0, 16, drain, 0)

    plsc.subcore_barrier()

    pltpu.sync_copy(cnt_sh, winv_v)

    @plsc.parallel_loop(0, _CNTP // 16, unroll=4)
    def wbody(i):
        cvec = winv_v[pl.ds(i * 16, 16)]
        winv_v[pl.ds(i * 16, 16)] = 1.0 / jnp.maximum(cvec, 1.0)

    pltpu.sync_copy(num_hbm, numbers_v)
    pltpu.sync_copy(gr_hbm, graph_v)

    lane = lax.iota(jnp.int32, 16)
    outs = (rn2_v, z_v, gi_v, w_v)
    out_hbms = (rn2_hbm, z_hbm, gi_hbm, w_hbm)

    def _load_descs(ch):
        p = lax.rem(ch, 2)
        base = wid * _EB + ch * _CH
        return (
            pltpu.make_async_copy(send_hbm.at[pl.ds(base, _CH)],
                                  send_v.at[pl.ds(p * _CH, _CH)], semL),
            pltpu.make_async_copy(recv_hbm.at[pl.ds(base, _CH)],
                                  recv_v.at[pl.ds(p * _CH, _CH)], semL),
            pltpu.make_async_copy(rv_hbm.at[pl.ds(3 * base, 3 * _CH)],
                                  rv_v.at[pl.ds(p * 3 * _CH, 3 * _CH)], semL),
        )

    def _store_descs(ch):
        p = lax.rem(ch, 2)
        base = wid * _EB + ch * _CH
        return tuple(
            pltpu.make_async_copy(v.at[pl.ds(p * _CH, _CH)],
                                  hbm.at[pl.ds(base, _CH)], semS)
            for v, hbm in zip(outs, out_hbms))

    for d in _load_descs(0):
        d.start()

    def chunk_body(ch, _):
        p = lax.rem(ch, 2)
        off0 = p * _CH
        for d in _load_descs(ch):
            d.wait()

        @pl.when(ch + 1 < _NCH)
        def _next_loads():
            for d in _load_descs(ch + 1):
                d.start()

        @pl.when(ch >= 2)
        def _wait_prev_stores():
            for d in _store_descs(ch - 2):
                d.wait()

        @plsc.parallel_loop(0, _VCH, unroll=4)
        def vbody(i):
            off = off0 + i * 16
            s16 = send_v[pl.ds(off, 16)]
            r16 = recv_v[pl.ds(off, 16)]
            z16 = plsc.load_gather(numbers_v, [s16])
            g16 = plsc.load_gather(graph_v, [r16])
            w16 = plsc.load_gather(winv_v, [r16])
            e3 = (off + lane) * 3
            x = plsc.load_gather(rv_v, [e3])
            y = plsc.load_gather(rv_v, [e3 + 1])
            zc = plsc.load_gather(rv_v, [e3 + 2])
            rn2_v[pl.ds(off, 16)] = x * x + y * y + zc * zc
            z_v[pl.ds(off, 16)] = z16.astype(jnp.float32)
            gi_v[pl.ds(off, 16)] = g16.astype(jnp.float32)
            w_v[pl.ds(off, 16)] = w16

        for d in _store_descs(ch):
            d.start()
        return 0

    lax.fori_loop(0, _NCH, chunk_body, 0)
    for q in (_NCH - 2, _NCH - 1):
        for d in _store_descs(q):
            d.wait()


_B = 5120
_NBLK = EP // _B


def _prep_body(wl0T, embTp, embWT_out):
    embWT_out[...] = jnp.dot(wl0T[...], embTp[...],
                             preferred_element_type=jnp.float32
                             ).astype(jnp.bfloat16)


def _tc_body(rn2_ref, zf_ref, gif_ref, w_ref, ugc, vgc, bc, mco, w2aT, b2ac,
             embWT_s, G_ref):
    i = pl.program_id(0)

    @pl.when(i == 0)
    def _init():
        G_ref[...] = jnp.zeros_like(G_ref)

    rn2 = rn2_ref[...].reshape(1, _B)
    zf = zf_ref[...].reshape(1, _B)
    gif = gif_ref[...].reshape(1, _B)
    w = w_ref[...].reshape(1, _B)
    rnorm = jnp.sqrt(rn2)
    mc = mco[...]
    var = rn2 * mc[0:1, 0:1] + rnorm * mc[0:1, 1:2] + mc[0:1, 2:3]
    inv = jax.lax.rsqrt(var + 1e-6)
    rni = rnorm * inv
    hn = rni * ugc[...] + (inv * vgc[...] + bc[...])
    a = hn * jax.nn.sigmoid(hn)
    scT = jnp.dot(w2aT[...], a.astype(jnp.bfloat16),
                  preferred_element_type=jnp.float32) + b2ac[...]
    zio = lax.broadcasted_iota(jnp.int32, (F, _B), 0)
    zoh = (zio == zf.astype(jnp.int32)).astype(jnp.bfloat16)
    embT = jnp.dot(embWT_s[...], zoh, preferred_element_type=jnp.float32)
    msgT = scT * embT
    gio = lax.broadcasted_iota(jnp.int32, (8, _B), 0)
    wq = jnp.where(gio == gif.astype(jnp.int32), w, 0.0)
    G_ref[...] += lax.dot_general(wq, msgT, (((1,), (1,)), ((), ())),
                                  preferred_element_type=jnp.float32)


def _final_body(G_ref, ngi_ref, m2w1, m2b1, m2g, m2bt, m2w2, m2b2, out_ref):
    ngi = ngi_ref[...]
    Gm = G_ref[...]
    rows = []
    for g in range(4):
        cg = jnp.maximum(jnp.sum((ngi == g).astype(jnp.float32)), 1.0)
        rows.append(Gm[g:g + 1, :] / cg)
    gmean = jnp.concatenate(rows, axis=0)
    h2 = jnp.dot(gmean, m2w1[...], preferred_element_type=jnp.float32) + m2b1[...]
    mu2 = jnp.mean(h2, axis=1, keepdims=True)
    var2 = jnp.mean((h2 - mu2) ** 2, axis=1, keepdims=True)
    hn2 = (h2 - mu2) / jnp.sqrt(var2 + 1e-6) * m2g[...] + m2bt[...]
    a2 = hn2 * jax.nn.sigmoid(hn2)
    out_ref[...] = jnp.dot(a2, m2w2[...],
                           preferred_element_type=jnp.float32) + m2b2[...]


def _const_spec(shape):
    return pl.BlockSpec(shape, lambda i: tuple(0 for _ in shape))


def _tc_call(rn2, zf, gif, wf, ugc, vgc, bc, mco, w2aT, b2ac, wl0T, embTp,
             ngi, m2w1, m2b1, m2g, m2bt, m2w2, m2b2, interpret=False):
    embWT = pl.pallas_call(
        _prep_body,
        out_shape=jax.ShapeDtypeStruct((F, F), jnp.bfloat16),
        interpret=interpret,
    )(wl0T, embTp)
    edge_spec = pl.BlockSpec((_B,), lambda i: (i,))
    G = pl.pallas_call(
        _tc_body,
        grid=(_NBLK,),
        in_specs=[
            edge_spec,
            edge_spec,
            edge_spec,
            edge_spec,
            _const_spec((HID, 1)),
            _const_spec((HID, 1)),
            _const_spec((HID, 1)),
            _const_spec((1, 4)),
            _const_spec((F, HID)),
            _const_spec((F, 1)),
            _const_spec((F, F)),
        ],
        out_specs=pl.BlockSpec((8, F), lambda i: (0, 0)),
        out_shape=jax.ShapeDtypeStruct((8, F), jnp.float32),
        compiler_params=pltpu.CompilerParams(
            dimension_semantics=("arbitrary",)),
        interpret=interpret,
    )(rn2, zf, gif, wf, ugc, vgc, bc, mco, w2aT, b2ac, embWT)
    return pl.pallas_call(
        _final_body,
        out_shape=jax.ShapeDtypeStruct((4, F), jnp.float32),
        interpret=interpret,
    )(G, ngi, m2w1, m2b1, m2g, m2bt, m2w2, m2b2)


def kernel(numbers, relative_vectors, senders, receivers, node_graph_idx,
           embed, Wl0, Wl1, m1_W1, m1_b1, m1_g, m1_beta, m1_W2, m1_b2,
           m2_W1, m2_b1, m2_g, m2_beta, m2_W2, m2_b2):
    f32, i32 = jnp.float32, jnp.int32
    rv_flat = relative_vectors.astype(f32).reshape(-1)
    rn2, zf, gif, wf = _sc_edge_streams_kernel()(
        rv_flat, senders.astype(i32), receivers.astype(i32),
        numbers.astype(i32), node_graph_idx.astype(i32))
    embTp = jnp.zeros((F, F), f32).at[:, :MAXZ].set(embed.T.astype(f32))
    ngi = jnp.concatenate(
        [node_graph_idx.astype(i32),
         jnp.full((80 * 128 - N,), 7, i32)]).reshape(80, 128)
    w1v = m1_W1.astype(f32).reshape(HID)
    b1v = m1_b1.astype(f32).reshape(HID)
    gv = m1_g.astype(f32).reshape(HID)
    uu = w1v - jnp.mean(w1v)
    vv = b1v - jnp.mean(b1v)
    mco = jnp.stack([jnp.mean(uu * uu), 2.0 * jnp.mean(uu * vv),
                     jnp.mean(vv * vv), jnp.float32(0.0)]).reshape(1, 4)
    return _tc_call(
        rn2, zf, gif, wf,
        (uu * gv).reshape(HID, 1),
        (vv * gv).reshape(HID, 1),
        m1_beta.astype(f32).reshape(HID, 1),
        mco,
        m1_W2[:, :F].T.astype(jnp.bfloat16),
        m1_b2[:F].astype(f32).reshape(F, 1),
        Wl0.T.astype(f32),
        embTp,
        ngi,
        m2_W1.astype(f32),
        m2_b1.astype(f32).reshape(1, HID),
        m2_g.astype(f32).reshape(1, HID),
        m2_beta.astype(f32).reshape(1, HID),
        m2_W2.astype(f32),
        m2_b2.astype(f32).reshape(1, F),
    )


# speedup vs baseline: 2.2300x; 2.2300x over previous
import functools

import jax
import jax.numpy as jnp
from jax import lax
from jax.experimental import pallas as pl
from jax.experimental.pallas import tpu as pltpu
from jax.experimental.pallas import tpu_sc as plsc

N = 10000
E = 320000
EP = 327680
F = 128
HID = 32
MAXZ = 119

_NS = 16
_NC = 2
_NTILE = _NS * _NC
_EA = E // _NS
_EAP = 20096
_AROWS = _EAP // 128
_EB = E // _NTILE
_CH = 2000
_NCH = _EB // _CH
_VCH = _CH // 16
_CNTP = N + 16
_DUMMY = N

@functools.cache
def _sc_edge_streams_kernel():
    mesh = plsc.VectorSubcoreMesh(core_axis_name="c", subcore_axis_name="s",
                                  num_cores=_NC, num_subcores=_NS)
    return pl.kernel(
        _sc_body,
        out_type=[jax.ShapeDtypeStruct((EP,), jnp.float32)] * 4,
        mesh=mesh,
        scratch_types=[
            pltpu.VMEM_SHARED((_CNTP,), jnp.float32),
            pltpu.VMEM((_EAP,), jnp.int32),
            pltpu.VMEM((_AROWS, 128), jnp.int32),
            pltpu.VMEM((128,), jnp.float32),
            pltpu.VMEM((_CNTP,), jnp.float32),
            pltpu.VMEM((N,), jnp.int32),
            pltpu.VMEM((N,), jnp.int32),
            pltpu.VMEM((2 * _CH,), jnp.int32),
            pltpu.VMEM((2 * _CH,), jnp.int32),
            pltpu.VMEM((2 * _CH,), jnp.float32),
            pltpu.VMEM((2 * _CH,), jnp.float32),
            pltpu.VMEM((2 * _CH,), jnp.float32),
            pltpu.VMEM((2 * _CH,), jnp.float32),
            pltpu.VMEM((EP - E,), jnp.float32),
            pltpu.SemaphoreType.DMA,
            pltpu.SemaphoreType.DMA,
            pltpu.SemaphoreType.DMA,
        ],
        compiler_params=pltpu.CompilerParams(needs_layout_passes=False),
    )


def _sc_body(rn2in_hbm, send_hbm, recv_hbm, num_hbm, gr_hbm,
                     rn2_hbm, z_hbm, gi_hbm, w_hbm,
                     cnt_sh, buf1d, idx2, ones_row, winv_v, numbers_v,
                     graph_v, send_v, recv_v, rn2_v, z_v, gi_v, w_v,
                     zpad_v, semA, semL, semS):
    c = lax.axis_index("c")
    s = lax.axis_index("s")
    wid = s * _NC + c

    ones16 = jnp.ones((16,), jnp.float32)
    for k in range(8):
        ones_row[pl.ds(k * 16, 16)] = ones16
    dummy16 = jnp.full((16,), _DUMMY, jnp.int32)
    for k in range((_EAP - _EA) // 16):
        buf1d[pl.ds(_EA + k * 16, 16)] = dummy16

    @pl.when(wid == 0)
    def _zero_tails():
        z16f = jnp.zeros((16,), jnp.float32)

        def zb(i, _):
            zpad_v[pl.ds(i * 16, 16)] = z16f
            return 0

        lax.fori_loop(0, (EP - E) // 16, zb, 0)
        pltpu.sync_copy(zpad_v, rn2_hbm.at[pl.ds(E, EP - E)])
        pltpu.sync_copy(zpad_v, z_hbm.at[pl.ds(E, EP - E)])
        pltpu.sync_copy(zpad_v, gi_hbm.at[pl.ds(E, EP - E)])
        pltpu.sync_copy(zpad_v, w_hbm.at[pl.ds(E, EP - E)])

    @pl.when(s == 0)
    def _zero_cnt():
        z16 = jnp.zeros((16,), jnp.float32)

        def zbody(i, _):
            winv_v[pl.ds(i * 16, 16)] = z16
            return 0

        lax.fori_loop(0, _CNTP // 16, zbody, 0)
        pltpu.sync_copy(winv_v, cnt_sh)

    pltpu.sync_copy(recv_hbm.at[pl.ds(s * _EA, _EA)], buf1d.at[pl.ds(0, _EA)])

    @plsc.parallel_loop(0, _EAP // 16, unroll=8)
    def repack(k):
        idx2[k // 8, pl.ds((k % 8) * 16, 16)] = buf1d[pl.ds(k * 16, 16)]

    plsc.subcore_barrier()

    def fire(j, _):
        pltpu.make_async_copy(ones_row, cnt_sh.at[idx2.at[j]], semA).start(add=True)

        @pl.when(j >= 16)
        def _():
            pltpu.make_async_copy(ones_row, cnt_sh.at[idx2.at[0]], semA).wait()

        return 0

    lax.fori_loop(0, _AROWS, fire, 0)

    def drain(j, _):
        pltpu.make_async_copy(ones_row, cnt_sh.at[idx2.at[0]], semA).wait()
        return 0

    lax.fori_loop(0, 16, drain, 0)

    plsc.subcore_barrier()

    pltpu.sync_copy(cnt_sh, winv_v)

    @plsc.parallel_loop(0, _CNTP // 16, unroll=4)
    def wbody(i):
        cvec = winv_v[pl.ds(i * 16, 16)]
        winv_v[pl.ds(i * 16, 16)] = 1.0 / jnp.maximum(cvec, 1.0)

    pltpu.sync_copy(num_hbm, numbers_v)
    pltpu.sync_copy(gr_hbm, graph_v)

    outs = (rn2_v, z_v, gi_v, w_v)
    out_hbms = (rn2_hbm, z_hbm, gi_hbm, w_hbm)

    def _load_descs(ch):
        p = lax.rem(ch, 2)
        base = wid * _EB + ch * _CH
        return (
            pltpu.make_async_copy(send_hbm.at[pl.ds(base, _CH)],
                                  send_v.at[pl.ds(p * _CH, _CH)], semL),
            pltpu.make_async_copy(recv_hbm.at[pl.ds(base, _CH)],
                                  recv_v.at[pl.ds(p * _CH, _CH)], semL),
            pltpu.make_async_copy(rn2in_hbm.at[pl.ds(base, _CH)],
                                  rn2_v.at[pl.ds(p * _CH, _CH)], semL),
        )

    def _store_descs(ch):
        p = lax.rem(ch, 2)
        base = wid * _EB + ch * _CH
        return tuple(
            pltpu.make_async_copy(v.at[pl.ds(p * _CH, _CH)],
                                  hbm.at[pl.ds(base, _CH)], semS)
            for v, hbm in zip(outs, out_hbms))

    for d in _load_descs(0):
        d.start()

    def chunk_body(ch, _):
        p = lax.rem(ch, 2)
        off0 = p * _CH
        for d in _load_descs(ch):
            d.wait()

        @pl.when(ch >= 1)
        def _wait_prev_stores():
            for d in _store_descs(ch - 1):
                d.wait()

        @pl.when(ch + 1 < _NCH)
        def _next_loads():
            for d in _load_descs(ch + 1):
                d.start()

        @plsc.parallel_loop(0, _VCH, unroll=4)
        def vbody(i):
            off = off0 + i * 16
            s16 = send_v[pl.ds(off, 16)]
            r16 = recv_v[pl.ds(off, 16)]
            z16 = plsc.load_gather(numbers_v, [s16])
            g16 = plsc.load_gather(graph_v, [r16])
            w16 = plsc.load_gather(winv_v, [r16])
            z_v[pl.ds(off, 16)] = z16.astype(jnp.float32)
            gi_v[pl.ds(off, 16)] = g16.astype(jnp.float32)
            w_v[pl.ds(off, 16)] = w16

        for d in _store_descs(ch):
            d.start()
        return 0

    lax.fori_loop(0, _NCH, chunk_body, 0)
    for d in _store_descs(_NCH - 1):
        d.wait()


_B = 5120
_NBLK = EP // _B


def _prep_body(wl0T, embTp, embWT_out):
    embWT_out[...] = jnp.dot(wl0T[...], embTp[...],
                             preferred_element_type=jnp.float32
                             ).astype(jnp.bfloat16)


def _tc_body(rn2_ref, zf_ref, gif_ref, w_ref, ugc, vgc, bc, mco, w2aT, b2ac,
             embWT_s, G_ref):
    i = pl.program_id(0)

    @pl.when(i == 0)
    def _init():
        G_ref[...] = jnp.zeros_like(G_ref)

    rn2 = rn2_ref[...].reshape(1, _B)
    zf = zf_ref[...].reshape(1, _B)
    gif = gif_ref[...].reshape(1, _B)
    w = w_ref[...].reshape(1, _B)
    rnorm = jnp.sqrt(rn2)
    mc = mco[...]
    var = rn2 * mc[0:1, 0:1] + rnorm * mc[0:1, 1:2] + mc[0:1, 2:3]
    inv = jax.lax.rsqrt(var + 1e-6)
    rni = rnorm * inv
    hn = rni * ugc[...] + (inv * vgc[...] + bc[...])
    a = hn * jax.nn.sigmoid(hn)
    scT = jnp.dot(w2aT[...], a.astype(jnp.bfloat16),
                  preferred_element_type=jnp.float32) + b2ac[...]
    zio = lax.broadcasted_iota(jnp.int32, (F, _B), 0)
    zoh = (zio == zf.astype(jnp.int32)).astype(jnp.bfloat16)
    embT = jnp.dot(embWT_s[...], zoh, preferred_element_type=jnp.float32)
    msgT = scT * embT
    gio = lax.broadcasted_iota(jnp.int32, (8, _B), 0)
    wq = jnp.where(gio == gif.astype(jnp.int32), w, 0.0)
    G_ref[...] += lax.dot_general(wq, msgT, (((1,), (1,)), ((), ())),
                                  preferred_element_type=jnp.float32)


def _final_body(G_ref, ngi_ref, m2w1, m2b1, m2g, m2bt, m2w2, m2b2, out_ref):
    ngi = ngi_ref[...]
    Gm = G_ref[...]
    rows = []
    for g in range(4):
        cg = jnp.maximum(jnp.sum((ngi == g).astype(jnp.float32)), 1.0)
        rows.append(Gm[g:g + 1, :] / cg)
    gmean = jnp.concatenate(rows, axis=0)
    h2 = jnp.dot(gmean, m2w1[...], preferred_element_type=jnp.float32) + m2b1[...]
    mu2 = jnp.mean(h2, axis=1, keepdims=True)
    var2 = jnp.mean((h2 - mu2) ** 2, axis=1, keepdims=True)
    hn2 = (h2 - mu2) / jnp.sqrt(var2 + 1e-6) * m2g[...] + m2bt[...]
    a2 = hn2 * jax.nn.sigmoid(hn2)
    out_ref[...] = jnp.dot(a2, m2w2[...],
                           preferred_element_type=jnp.float32) + m2b2[...]


def _const_spec(shape):
    return pl.BlockSpec(shape, lambda i: tuple(0 for _ in shape))


def _tc_call(rn2, zf, gif, wf, ugc, vgc, bc, mco, w2aT, b2ac, wl0T, embTp,
             ngi, m2w1, m2b1, m2g, m2bt, m2w2, m2b2, interpret=False):
    embWT = pl.pallas_call(
        _prep_body,
        out_shape=jax.ShapeDtypeStruct((F, F), jnp.bfloat16),
        interpret=interpret,
    )(wl0T, embTp)
    edge_spec = pl.BlockSpec((_B,), lambda i: (i,))
    G = pl.pallas_call(
        _tc_body,
        grid=(_NBLK,),
        in_specs=[
            edge_spec,
            edge_spec,
            edge_spec,
            edge_spec,
            _const_spec((HID, 1)),
            _const_spec((HID, 1)),
            _const_spec((HID, 1)),
            _const_spec((1, 4)),
            _const_spec((F, HID)),
            _const_spec((F, 1)),
            _const_spec((F, F)),
        ],
        out_specs=pl.BlockSpec((8, F), lambda i: (0, 0)),
        out_shape=jax.ShapeDtypeStruct((8, F), jnp.float32),
        compiler_params=pltpu.CompilerParams(
            dimension_semantics=("arbitrary",)),
        interpret=interpret,
    )(rn2, zf, gif, wf, ugc, vgc, bc, mco, w2aT, b2ac, embWT)
    return pl.pallas_call(
        _final_body,
        out_shape=jax.ShapeDtypeStruct((4, F), jnp.float32),
        interpret=interpret,
    )(G, ngi, m2w1, m2b1, m2g, m2bt, m2w2, m2b2)


def kernel(numbers, relative_vectors, senders, receivers, node_graph_idx,
           embed, Wl0, Wl1, m1_W1, m1_b1, m1_g, m1_beta, m1_W2, m1_b2,
           m2_W1, m2_b1, m2_g, m2_beta, m2_W2, m2_b2):
    f32, i32 = jnp.float32, jnp.int32
    rv32 = relative_vectors.astype(f32)
    rn2_full = jnp.sum(rv32 * rv32, axis=1)
    rn2, zf, gif, wf = _sc_edge_streams_kernel()(
        rn2_full, senders.astype(i32), receivers.astype(i32),
        numbers.astype(i32), node_graph_idx.astype(i32))
    embTp = jnp.zeros((F, F), f32).at[:, :MAXZ].set(embed.T.astype(f32))
    ngi = jnp.concatenate(
        [node_graph_idx.astype(i32),
         jnp.full((80 * 128 - N,), 7, i32)]).reshape(80, 128)
    w1v = m1_W1.astype(f32).reshape(HID)
    b1v = m1_b1.astype(f32).reshape(HID)
    gv = m1_g.astype(f32).reshape(HID)
    uu = w1v - jnp.mean(w1v)
    vv = b1v - jnp.mean(b1v)
    mco = jnp.stack([jnp.mean(uu * uu), 2.0 * jnp.mean(uu * vv),
                     jnp.mean(vv * vv), jnp.float32(0.0)]).reshape(1, 4)
    return _tc_call(
        rn2, zf, gif, wf,
        (uu * gv).reshape(HID, 1),
        (vv * gv).reshape(HID, 1),
        m1_beta.astype(f32).reshape(HID, 1),
        mco,
        m1_W2[:, :F].T.astype(jnp.bfloat16),
        m1_b2[:F].astype(f32).reshape(F, 1),
        Wl0.T.astype(f32),
        embTp,
        ngi,
        m2_W1.astype(f32),
        m2_b1.astype(f32).reshape(1, HID),
        m2_g.astype(f32).reshape(1, HID),
        m2_beta.astype(f32).reshape(1, HID),
        m2_W2.astype(f32),
        m2_b2.astype(f32).reshape(1, F),
    )


# speedup vs baseline: 2.4342x; 1.0916x over previous
import functools

import jax
import jax.numpy as jnp
from jax import lax
from jax.experimental import pallas as pl
from jax.experimental.pallas import tpu as pltpu
from jax.experimental.pallas import tpu_sc as plsc

N = 10000
E = 320000
EP = 327680
F = 128
HID = 32
MAXZ = 119

_NS = 16
_NC = 2
_NTILE = _NS * _NC
_EA = E // _NS
_EAP = 20096
_AROWS = _EAP // 128
_EB = E // _NTILE
_CH = 2000
_NCH = _EB // _CH
_VCH = _CH // 16
_CNTP = N + 16
_DUMMY = N

@functools.cache
def _sc_edge_streams_kernel():
    mesh = plsc.VectorSubcoreMesh(core_axis_name="c", subcore_axis_name="s",
                                  num_cores=_NC, num_subcores=_NS)
    return pl.kernel(
        _sc_body,
        out_type=[jax.ShapeDtypeStruct((EP,), jnp.float32)] * 3,
        mesh=mesh,
        scratch_types=[
            pltpu.VMEM_SHARED((_CNTP,), jnp.float32),
            pltpu.VMEM((_EAP,), jnp.int32),
            pltpu.VMEM((_AROWS, 128), jnp.int32),
            pltpu.VMEM((128,), jnp.float32),
            pltpu.VMEM((_CNTP,), jnp.float32),
            pltpu.VMEM((N,), jnp.int32),
            pltpu.VMEM((N,), jnp.int32),
            pltpu.VMEM((2 * _CH,), jnp.int32),
            pltpu.VMEM((2 * _CH,), jnp.int32),
            pltpu.VMEM((2 * _CH,), jnp.float32),
            pltpu.VMEM((2 * _CH,), jnp.float32),
            pltpu.VMEM((2 * _CH,), jnp.float32),
            pltpu.VMEM((EP - E,), jnp.float32),
            pltpu.SemaphoreType.DMA,
            pltpu.SemaphoreType.DMA,
            pltpu.SemaphoreType.DMA,
        ],
        compiler_params=pltpu.CompilerParams(needs_layout_passes=False),
    )


def _sc_body(send_hbm, recv_hbm, num_hbm, gr_hbm,
                     z_hbm, gi_hbm, w_hbm,
                     cnt_sh, buf1d, idx2, ones_row, winv_v, numbers_v,
                     graph_v, send_v, recv_v, z_v, gi_v, w_v,
                     zpad_v, semA, semL, semS):
    c = lax.axis_index("c")
    s = lax.axis_index("s")
    wid = s * _NC + c

    ones16 = jnp.ones((16,), jnp.float32)
    for k in range(8):
        ones_row[pl.ds(k * 16, 16)] = ones16
    dummy16 = jnp.full((16,), _DUMMY, jnp.int32)
    for k in range((_EAP - _EA) // 16):
        buf1d[pl.ds(_EA + k * 16, 16)] = dummy16

    @pl.when(wid == 0)
    def _zero_tails():
        z16f = jnp.zeros((16,), jnp.float32)

        def zb(i, _):
            zpad_v[pl.ds(i * 16, 16)] = z16f
            return 0

        lax.fori_loop(0, (EP - E) // 16, zb, 0)
        pltpu.sync_copy(zpad_v, z_hbm.at[pl.ds(E, EP - E)])
        pltpu.sync_copy(zpad_v, gi_hbm.at[pl.ds(E, EP - E)])
        pltpu.sync_copy(zpad_v, w_hbm.at[pl.ds(E, EP - E)])

    @pl.when(s == 0)
    def _zero_cnt():
        z16 = jnp.zeros((16,), jnp.float32)

        def zbody(i, _):
            winv_v[pl.ds(i * 16, 16)] = z16
            return 0

        lax.fori_loop(0, _CNTP // 16, zbody, 0)
        pltpu.sync_copy(winv_v, cnt_sh)

    pltpu.sync_copy(recv_hbm.at[pl.ds(s * _EA, _EA)], buf1d.at[pl.ds(0, _EA)])

    @plsc.parallel_loop(0, _EAP // 16, unroll=8)
    def repack(k):
        idx2[k // 8, pl.ds((k % 8) * 16, 16)] = buf1d[pl.ds(k * 16, 16)]

    plsc.subcore_barrier()

    def fire(j, _):
        pltpu.make_async_copy(ones_row, cnt_sh.at[idx2.at[j]], semA).start(add=True)

        @pl.when(j >= 16)
        def _():
            pltpu.make_async_copy(ones_row, cnt_sh.at[idx2.at[0]], semA).wait()

        return 0

    lax.fori_loop(0, _AROWS, fire, 0)

    def drain(j, _):
        pltpu.make_async_copy(ones_row, cnt_sh.at[idx2.at[0]], semA).wait()
        return 0

    lax.fori_loop(0, 16, drain, 0)

    plsc.subcore_barrier()

    pltpu.sync_copy(cnt_sh, winv_v)

    @plsc.parallel_loop(0, _CNTP // 16, unroll=4)
    def wbody(i):
        cvec = winv_v[pl.ds(i * 16, 16)]
        winv_v[pl.ds(i * 16, 16)] = 1.0 / jnp.maximum(cvec, 1.0)

    pltpu.sync_copy(num_hbm, numbers_v)
    pltpu.sync_copy(gr_hbm, graph_v)

    outs = (z_v, gi_v, w_v)
    out_hbms = (z_hbm, gi_hbm, w_hbm)

    def _load_descs(ch):
        p = lax.rem(ch, 2)
        base = wid * _EB + ch * _CH
        return (
            pltpu.make_async_copy(send_hbm.at[pl.ds(base, _CH)],
                                  send_v.at[pl.ds(p * _CH, _CH)], semL),
            pltpu.make_async_copy(recv_hbm.at[pl.ds(base, _CH)],
                                  recv_v.at[pl.ds(p * _CH, _CH)], semL),
        )

    def _store_descs(ch):
        p = lax.rem(ch, 2)
        base = wid * _EB + ch * _CH
        return tuple(
            pltpu.make_async_copy(v.at[pl.ds(p * _CH, _CH)],
                                  hbm.at[pl.ds(base, _CH)], semS)
            for v, hbm in zip(outs, out_hbms))

    for d in _load_descs(0):
        d.start()

    def chunk_body(ch, _):
        p = lax.rem(ch, 2)
        off0 = p * _CH
        for d in _load_descs(ch):
            d.wait()

        @pl.when(ch >= 1)
        def _wait_prev_stores():
            for d in _store_descs(ch - 1):
                d.wait()

        @pl.when(ch + 1 < _NCH)
        def _next_loads():
            for d in _load_descs(ch + 1):
                d.start()

        @plsc.parallel_loop(0, _VCH, unroll=4)
        def vbody(i):
            off = off0 + i * 16
            s16 = send_v[pl.ds(off, 16)]
            r16 = recv_v[pl.ds(off, 16)]
            z16 = plsc.load_gather(numbers_v, [s16])
            g16 = plsc.load_gather(graph_v, [r16])
            w16 = plsc.load_gather(winv_v, [r16])
            z_v[pl.ds(off, 16)] = z16.astype(jnp.float32)
            gi_v[pl.ds(off, 16)] = g16.astype(jnp.float32)
            w_v[pl.ds(off, 16)] = w16

        for d in _store_descs(ch):
            d.start()
        return 0

    lax.fori_loop(0, _NCH, chunk_body, 0)
    for d in _store_descs(_NCH - 1):
        d.wait()


_B = 5120
_NBLK = EP // _B


def _prep_body(wl0T, embTp, embWT_out):
    embWT_out[...] = jnp.dot(wl0T[...], embTp[...],
                             preferred_element_type=jnp.float32
                             ).astype(jnp.bfloat16)


def _tc_body(rn2_ref, zf_ref, gif_ref, w_ref, ugc, vgc, bc, mco, w2aT, b2ac,
             embWT_s, G_ref):
    i = pl.program_id(0)

    @pl.when(i == 0)
    def _init():
        G_ref[...] = jnp.zeros_like(G_ref)

    rn2 = rn2_ref[...].reshape(1, _B)
    zf = zf_ref[...].reshape(1, _B)
    gif = gif_ref[...].reshape(1, _B)
    w = w_ref[...].reshape(1, _B)
    rnorm = jnp.sqrt(rn2)
    mc = mco[...]
    var = rn2 * mc[0:1, 0:1] + rnorm * mc[0:1, 1:2] + mc[0:1, 2:3]
    inv = jax.lax.rsqrt(var + 1e-6)
    rni = rnorm * inv
    hn = rni * ugc[...] + (inv * vgc[...] + bc[...])
    a = hn * jax.nn.sigmoid(hn)
    scT = jnp.dot(w2aT[...], a.astype(jnp.bfloat16),
                  preferred_element_type=jnp.float32) + b2ac[...]
    zio = lax.broadcasted_iota(jnp.int32, (F, _B), 0)
    zoh = (zio == zf.astype(jnp.int32)).astype(jnp.bfloat16)
    embT = jnp.dot(embWT_s[...], zoh, preferred_element_type=jnp.float32)
    msgT = scT * embT
    gio = lax.broadcasted_iota(jnp.int32, (8, _B), 0)
    wq = jnp.where(gio == gif.astype(jnp.int32), w, 0.0)
    G_ref[...] += lax.dot_general(wq, msgT, (((1,), (1,)), ((), ())),
                                  preferred_element_type=jnp.float32)


def _final_body(G_ref, ngi_ref, m2w1, m2b1, m2g, m2bt, m2w2, m2b2, out_ref):
    ngi = ngi_ref[...]
    Gm = G_ref[...]
    rows = []
    for g in range(4):
        cg = jnp.maximum(jnp.sum((ngi == g).astype(jnp.float32)), 1.0)
        rows.append(Gm[g:g + 1, :] / cg)
    gmean = jnp.concatenate(rows, axis=0)
    h2 = jnp.dot(gmean, m2w1[...], preferred_element_type=jnp.float32) + m2b1[...]
    mu2 = jnp.mean(h2, axis=1, keepdims=True)
    var2 = jnp.mean((h2 - mu2) ** 2, axis=1, keepdims=True)
    hn2 = (h2 - mu2) / jnp.sqrt(var2 + 1e-6) * m2g[...] + m2bt[...]
    a2 = hn2 * jax.nn.sigmoid(hn2)
    out_ref[...] = jnp.dot(a2, m2w2[...],
                           preferred_element_type=jnp.float32) + m2b2[...]


def _const_spec(shape):
    return pl.BlockSpec(shape, lambda i: tuple(0 for _ in shape))


def _tc_call(rn2, zf, gif, wf, ugc, vgc, bc, mco, w2aT, b2ac, wl0T, embTp,
             ngi, m2w1, m2b1, m2g, m2bt, m2w2, m2b2, interpret=False):
    embWT = pl.pallas_call(
        _prep_body,
        out_shape=jax.ShapeDtypeStruct((F, F), jnp.bfloat16),
        interpret=interpret,
    )(wl0T, embTp)
    edge_spec = pl.BlockSpec((_B,), lambda i: (i,))
    G = pl.pallas_call(
        _tc_body,
        grid=(_NBLK,),
        in_specs=[
            edge_spec,
            edge_spec,
            edge_spec,
            edge_spec,
            _const_spec((HID, 1)),
            _const_spec((HID, 1)),
            _const_spec((HID, 1)),
            _const_spec((1, 4)),
            _const_spec((F, HID)),
            _const_spec((F, 1)),
            _const_spec((F, F)),
        ],
        out_specs=pl.BlockSpec((8, F), lambda i: (0, 0)),
        out_shape=jax.ShapeDtypeStruct((8, F), jnp.float32),
        compiler_params=pltpu.CompilerParams(
            dimension_semantics=("arbitrary",)),
        interpret=interpret,
    )(rn2, zf, gif, wf, ugc, vgc, bc, mco, w2aT, b2ac, embWT)
    return pl.pallas_call(
        _final_body,
        out_shape=jax.ShapeDtypeStruct((4, F), jnp.float32),
        interpret=interpret,
    )(G, ngi, m2w1, m2b1, m2g, m2bt, m2w2, m2b2)


def kernel(numbers, relative_vectors, senders, receivers, node_graph_idx,
           embed, Wl0, Wl1, m1_W1, m1_b1, m1_g, m1_beta, m1_W2, m1_b2,
           m2_W1, m2_b1, m2_g, m2_beta, m2_W2, m2_b2):
    f32, i32 = jnp.float32, jnp.int32
    rv32 = relative_vectors.astype(f32)
    rn2 = jnp.pad(jnp.sum(rv32 * rv32, axis=1), (0, EP - E))
    zf, gif, wf = _sc_edge_streams_kernel()(
        senders.astype(i32), receivers.astype(i32),
        numbers.astype(i32), node_graph_idx.astype(i32))
    embTp = jnp.zeros((F, F), f32).at[:, :MAXZ].set(embed.T.astype(f32))
    ngi = jnp.concatenate(
        [node_graph_idx.astype(i32),
         jnp.full((80 * 128 - N,), 7, i32)]).reshape(80, 128)
    w1v = m1_W1.astype(f32).reshape(HID)
    b1v = m1_b1.astype(f32).reshape(HID)
    gv = m1_g.astype(f32).reshape(HID)
    uu = w1v - jnp.mean(w1v)
    vv = b1v - jnp.mean(b1v)
    mco = jnp.stack([jnp.mean(uu * uu), 2.0 * jnp.mean(uu * vv),
                     jnp.mean(vv * vv), jnp.float32(0.0)]).reshape(1, 4)
    return _tc_call(
        rn2, zf, gif, wf,
        (uu * gv).reshape(HID, 1),
        (vv * gv).reshape(HID, 1),
        m1_beta.astype(f32).reshape(HID, 1),
        mco,
        m1_W2[:, :F].T.astype(jnp.bfloat16),
        m1_b2[:F].astype(f32).reshape(F, 1),
        Wl0.T.astype(f32),
        embTp,
        ngi,
        m2_W1.astype(f32),
        m2_b1.astype(f32).reshape(1, HID),
        m2_g.astype(f32).reshape(1, HID),
        m2_beta.astype(f32).reshape(1, HID),
        m2_W2.astype(f32),
        m2_b2.astype(f32).reshape(1, F),
    )
